# baseline (speedup 1.0000x reference)
import functools

import jax
import jax.numpy as jnp
from jax import lax
from jax.experimental import pallas as pl
from jax.experimental.pallas import tpu as pltpu
from jax.experimental.pallas import tpu_sc as plsc

VOCAB = 100000
HID = 768
MAXPOS = 4096
PAD = 1
EPS = 1e-12
B = 4
S = 2048

NC = 2
NS = 16
L = 16
NW = NC * NS
N = B * S
TPW = N // NW
CH = 32
NCHUNK = TPW // CH
NJ = HID // L


def _rsqrt(x):
    i = plsc.bitcast(x, jnp.int32)
    i = jnp.int32(0x5F3759DF) - (i >> 1)
    y = plsc.bitcast(i, jnp.float32)
    for _ in range(3):
        y = y * (1.5 - 0.5 * x * y * y)
    return y


def _body(ids_hbm, word_hbm, pos_hbm, gamma_hbm, beta_hbm, out_hbm,
          ids_v, pidx_v, pf_v, wbuf, pbuf, gbuf, bbuf, semw, semp):
    wid = lax.axis_index("c") * NS + lax.axis_index("s")
    tok_base = wid * TPW
    row_start = (tok_base // S) * S
    npre = (tok_base - row_start) // TPW

    pltpu.sync_copy(ids_hbm.at[pl.ds(tok_base, TPW)], ids_v)

    def pf_body(k, acc):
        pltpu.sync_copy(ids_hbm.at[pl.ds(row_start + k * TPW, TPW)], pf_v)
        for i in range(TPW // L):
            v = pf_v[pl.ds(i * L, L)]
            acc = acc + jnp.where(v != PAD, 1, 0).astype(jnp.int32)
        return acc
    accv = lax.fori_loop(0, npre, pf_body, jnp.zeros((L,), jnp.int32))
    cnt = jnp.sum(accv)

    for i in range(TPW // L):
        v = ids_v[pl.ds(i * L, L)]
        m = jnp.where(v != PAD, 1, 0).astype(jnp.int32)
        s = plsc.cumsum(m)
        pidx_v[pl.ds(i * L, L)] = (s + cnt) * m + PAD
        cnt = cnt + jnp.sum(m)

    pltpu.sync_copy(gamma_hbm, gbuf)
    pltpu.sync_copy(beta_hbm, bbuf)

    rows0 = lax.iota(jnp.int32, L)
    rows1 = rows0 + L
    inv = jnp.float32(1.0 / HID)

    def chunk_body(c, _):
        cw = pltpu.async_copy(word_hbm.at[ids_v.at[pl.ds(c * CH, CH)]],
                              wbuf, semw)
        cp = pltpu.async_copy(pos_hbm.at[pidx_v.at[pl.ds(c * CH, CH)]],
                              pbuf, semp)
        cw.wait()
        cp.wait()

        def p1(j, carry):
            s0, q0, s1, q1 = carry
            jv = jnp.broadcast_to(j, (L,)).astype(jnp.int32)
            w0 = plsc.load_gather(wbuf, [rows0, jv])
            p0 = plsc.load_gather(pbuf, [rows0, jv])
            e0 = w0 + p0
            plsc.store_scatter(wbuf, [rows0, jv], e0)
            w1 = plsc.load_gather(wbuf, [rows1, jv])
            p1v = plsc.load_gather(pbuf, [rows1, jv])
            e1 = w1 + p1v
            plsc.store_scatter(wbuf, [rows1, jv], e1)
            return (s0 + e0, q0 + e0 * e0, s1 + e1, q1 + e1 * e1)

        z = jnp.zeros((L,), jnp.float32)
        s0, q0, s1, q1 = lax.fori_loop(0, HID, p1, (z, z, z, z))

        mu0 = s0 * inv
        mu1 = s1 * inv
        r0 = _rsqrt(q0 * inv - mu0 * mu0 + EPS)
        r1 = _rsqrt(q1 * inv - mu1 * mu1 + EPS)

        def p2(j, carry):
            jv = jnp.broadcast_to(j, (L,)).astype(jnp.int32)
            g = plsc.load_gather(gbuf, [jv])
            b = plsc.load_gather(bbuf, [jv])
            e0 = plsc.load_gather(wbuf, [rows0, jv])
            plsc.store_scatter(pbuf, [rows0, jv], (e0 - mu0) * r0 * g + b)
            e1 = plsc.load_gather(wbuf, [rows1, jv])
            plsc.store_scatter(pbuf, [rows1, jv], (e1 - mu1) * r1 * g + b)
            return carry

        lax.fori_loop(0, HID, p2, 0)
        pltpu.sync_copy(pbuf, out_hbm.at[pl.ds(tok_base + c * CH, CH)])
        return 0

    lax.fori_loop(0, NCHUNK, chunk_body, 0)


@jax.jit
def _run(ids_flat, word_table, pos_table, gamma, beta):
    mesh = plsc.VectorSubcoreMesh(core_axis_name="c", subcore_axis_name="s",
                                  num_cores=NC, num_subcores=NS)
    kern = pl.kernel(
        _body,
        out_type=jax.ShapeDtypeStruct((N, HID), jnp.float32),
        mesh=mesh,
        compiler_params=pltpu.CompilerParams(needs_layout_passes=False),
        scratch_types=[
            pltpu.VMEM((TPW,), jnp.int32),
            pltpu.VMEM((TPW,), jnp.int32),
            pltpu.VMEM((TPW,), jnp.int32),
            pltpu.VMEM((CH, HID), jnp.float32),
            pltpu.VMEM((CH, HID), jnp.float32),
            pltpu.VMEM((HID,), jnp.float32),
            pltpu.VMEM((HID,), jnp.float32),
            pltpu.SemaphoreType.DMA,
            pltpu.SemaphoreType.DMA,
        ],
    )
    return kern(ids_flat, word_table, pos_table, gamma, beta)


def kernel(input_ids, word_table, pos_table, gamma, beta):
    ids_flat = input_ids.reshape(-1).astype(jnp.int32)
    out = _run(ids_flat, word_table, pos_table, gamma, beta)
    return out.reshape(B, S, HID)
